# Initial kernel scaffold; baseline (speedup 1.0000x reference)
import jax
import jax.numpy as jnp
from jax.experimental import pallas as pl


def kernel(x_pre, A, temperature):
    raise NotImplementedError("write your pallas kernel here")


# speedup vs baseline: 8.7354x; 8.7354x over previous
import jax
import jax.numpy as jnp
from jax.experimental import pallas as pl
from jax.experimental.pallas import tpu as pltpu

K = 16
N = 512
T = 8
F = 128


def _tc_body(t_ref, yt_ref, logp_ref, edges_ref):
    temp = jnp.exp(jnp.clip(t_ref[0, 0], -5.0, 5.0))
    Yt = yt_ref[...]

    S_row = jnp.sum(Yt * Yt, axis=0, keepdims=True)
    ones_col = jnp.ones((T * F, 1), dtype=jnp.float32)
    S_col = jax.lax.dot_general(Yt * Yt, ones_col, (((0,), (0,)), ((), ())),
                                preferred_element_type=jnp.float32)
    M = jax.lax.dot_general(Yt, Yt, (((0,), (0,)), ((), ())),
                            preferred_element_type=jnp.float32)
    lq = temp * (S_col + S_row - 2.0 * M)

    lq2 = lq * lq
    sn_col = jnp.sum(lq2, axis=1, keepdims=True)
    sn_row = jnp.sum(lq2, axis=0, keepdims=True)
    C = jax.lax.dot_general(lq, lq, (((1,), (1,)), ((), ())),
                            preferred_element_type=jnp.float32)
    score = jnp.maximum(sn_col + sn_row - 2.0 * C, 0.0)

    lane = jax.lax.broadcasted_iota(jnp.int32, (N, N), 1)
    idx_cols = []
    for _ in range(K):
        m = jnp.min(score, axis=1, keepdims=True)
        am = jnp.min(jnp.where(score == m, lane, jnp.int32(1 << 20)),
                     axis=1, keepdims=True)
        idx_cols.append(am)
        score = jnp.where(lane == am, jnp.float32(jnp.inf), score)

    X0t = Yt[0:F, :]
    n0_row = jnp.sum(X0t * X0t, axis=0, keepdims=True)
    ones_f = jnp.ones((F, 1), dtype=jnp.float32)
    row_iota = jax.lax.broadcasted_iota(jnp.int32, (N, 1), 0)
    for i in range(T):
        Xit = Yt[F * i:F * (i + 1), :]
        Gi = jax.lax.dot_general(Xit, X0t, (((0,), (0,)), ((), ())),
                                 preferred_element_type=jnp.float32)
        ni_col = jax.lax.dot_general(Xit * Xit, ones_f, (((0,), (0,)), ((), ())),
                                     preferred_element_type=jnp.float32)
        LPi = (-temp) * (n0_row + ni_col - 2.0 * Gi)
        off = jnp.int32(N * i)
        for j in range(K):
            am = idx_cols[j]
            sel = jnp.where(lane == am, LPi, 0.0)
            logp_ref[i, :, pl.ds(j, 1)] = jnp.sum(sel, axis=1, keepdims=True)
            edges_ref[0, i, :, pl.ds(j, 1)] = row_iota + off
            edges_ref[1, i, :, pl.ds(j, 1)] = am + off


def kernel(x_pre, A, temperature):
    del A
    b, t, n, f = x_pre.shape
    xs = x_pre[0]
    Yt = jnp.transpose(xs, (0, 2, 1)).reshape(t * f, n)
    t_arr = jnp.reshape(temperature.astype(jnp.float32), (1, 1))

    logp, edges = pl.pallas_call(
        _tc_body,
        out_shape=[
            jax.ShapeDtypeStruct((T, N, K), jnp.float32),
            jax.ShapeDtypeStruct((2, T, N, K), jnp.int32),
        ],
        in_specs=[
            pl.BlockSpec(memory_space=pltpu.SMEM),
            pl.BlockSpec(memory_space=pltpu.VMEM),
        ],
        out_specs=[
            pl.BlockSpec(memory_space=pltpu.VMEM),
            pl.BlockSpec(memory_space=pltpu.VMEM),
        ],
    )(t_arr, Yt)

    return (x_pre, edges.reshape(2, t * n * K), logp)
